# baseline (speedup 1.0000x reference)
import functools

import jax
import jax.numpy as jnp
from jax import lax
from jax.experimental import pallas as pl
from jax.experimental.pallas import tpu as pltpu, tpu_sc as plsc

N = 10000
E = 320000
D = 128
H = 256
G = 64
C = 10

NC = 2
NS = 16
NW = NC * NS
CHUNK = 80
NCHUNK = 128
EPW = NCHUNK * CHUNK
EPAD = NW * EPW
NBUF = 4
NIB = 16
NGRP = NCHUNK // NIB
NPAD = 10240
RPT = NPAD // NS


def _segsum_body(h_hbm, eidx_hbm, zeros_hbm, out_hbm,
                 idx_v, rows_v, shared, *sems):
    isems = sems[:NIB]
    gsems = sems[NIB:NIB + NBUF]
    ssems = sems[NIB + NBUF:]
    c = lax.axis_index("c")
    s = lax.axis_index("s")
    w = c * NS + s
    pltpu.sync_copy(zeros_hbm, shared.at[pl.ds(s * RPT, RPT)])
    for q in range(NIB):
        pltpu.async_copy(eidx_hbm.at[w, q], idx_v.at[q], isems[q])
    for k in range(2):
        pltpu.make_async_copy(eidx_hbm.at[w, k], idx_v.at[k], isems[k]).wait()
        pltpu.async_copy(h_hbm.at[idx_v.at[k, 0]], rows_v.at[k], gsems[k])
    plsc.subcore_barrier()

    def group(g, carry):
        for u in range(NIB):
            j = g * NIB + u
            b = u % NBUF
            b2 = (u + 2) % NBUF
            q2 = (u + 2) % NIB
            qf = (u + 14) % NIB
            pltpu.make_async_copy(h_hbm.at[idx_v.at[u, 0]], rows_v.at[b],
                                  gsems[b]).wait()
            pltpu.async_copy(rows_v.at[b], shared.at[idx_v.at[u, 1]],
                             ssems[b], add=True)

            @pl.when(j >= 2)
            def _():
                pltpu.make_async_copy(rows_v.at[b2],
                                      shared.at[idx_v.at[qf, 1]],
                                      ssems[b2]).wait()

            @pl.when(jnp.logical_and(j >= 2, j + 14 < NCHUNK))
            def _():
                pltpu.async_copy(eidx_hbm.at[w, j + 14], idx_v.at[qf],
                                 isems[qf])

            @pl.when(j + 2 < NCHUNK)
            def _():
                pltpu.make_async_copy(eidx_hbm.at[w, j + 2], idx_v.at[q2],
                                      isems[q2]).wait()
                pltpu.async_copy(h_hbm.at[idx_v.at[q2, 0]], rows_v.at[b2],
                                 gsems[b2])
        return carry

    lax.fori_loop(0, NGRP, group, 0)
    for j in (NCHUNK - 2, NCHUNK - 1):
        b = j % NBUF
        pltpu.make_async_copy(rows_v.at[b], shared.at[idx_v.at[j % NIB, 1]],
                              ssems[b]).wait()
    plsc.subcore_barrier()
    pltpu.sync_copy(shared.at[pl.ds(s * RPT, RPT)],
                    out_hbm.at[c, pl.ds(s * RPT, RPT)])


def _segsum(h, eidx, zeros):
    mesh = plsc.VectorSubcoreMesh(core_axis_name="c", subcore_axis_name="s")
    f = pl.kernel(
        _segsum_body,
        mesh=mesh,
        out_type=jax.ShapeDtypeStruct((NC, NPAD, D), jnp.float32),
        scratch_types=[
            pltpu.VMEM((NIB, 2, CHUNK), jnp.int32),
            pltpu.VMEM((NBUF, CHUNK, D), jnp.float32),
            pltpu.VMEM_SHARED((NPAD, D), jnp.float32),
        ] + [pltpu.SemaphoreType.DMA] * (NIB + 2 * NBUF),
    )
    return f(h, eidx, zeros)


BLK = 2000


def _mlp_compute(h_ref, a_ref, W1_ref, b1_ref, W2_ref, b2_ref, relu_out):
    z = h_ref[...] + a_ref[0] + a_ref[1]
    z = jnp.dot(z, W1_ref[...], preferred_element_type=jnp.float32,
                precision=lax.Precision.HIGHEST) + b1_ref[...]
    z = jnp.maximum(z, 0.0)
    z = jnp.dot(z, W2_ref[...], preferred_element_type=jnp.float32,
                precision=lax.Precision.HIGHEST) + b2_ref[...]
    if relu_out:
        z = jnp.maximum(z, 0.0)
    return z


def _mlp_body(h_ref, a_ref, W1_ref, b1_ref, W2_ref, b2_ref, o_ref):
    o_ref[...] = _mlp_compute(h_ref, a_ref, W1_ref, b1_ref, W2_ref, b2_ref,
                              relu_out=True)


def _mlp(h, agg, W1, b1, W2, b2):
    return pl.pallas_call(
        _mlp_body,
        grid=(N // BLK,),
        in_specs=[
            pl.BlockSpec((BLK, D), lambda i: (i, 0)),
            pl.BlockSpec((NC, BLK, D), lambda i: (0, i, 0)),
            pl.BlockSpec((D, H), lambda i: (0, 0)),
            pl.BlockSpec((1, H), lambda i: (0, 0)),
            pl.BlockSpec((H, D), lambda i: (0, 0)),
            pl.BlockSpec((1, D), lambda i: (0, 0)),
        ],
        out_specs=pl.BlockSpec((BLK, D), lambda i: (i, 0)),
        out_shape=jax.ShapeDtypeStruct((N, D), jnp.float32),
    )(h, agg, W1, b1, W2, b2)


def _mlp_pool_body(h_ref, a_ref, batch_ref, W1_ref, b1_ref, W2_ref, b2_ref,
                   cw_ref, cb_ref, o_ref, sums_ref, cnt_ref):
    i = pl.program_id(0)

    @pl.when(i == 0)
    def _():
        sums_ref[...] = jnp.zeros_like(sums_ref)
        cnt_ref[...] = jnp.zeros_like(cnt_ref)

    z = _mlp_compute(h_ref, a_ref, W1_ref, b1_ref, W2_ref, b2_ref,
                     relu_out=False)
    onehot = (batch_ref[...] ==
              lax.broadcasted_iota(jnp.int32, (BLK, G), 1)).astype(jnp.float32)
    dn = (((0,), (0,)), ((), ()))
    sums_ref[...] += lax.dot_general(onehot, z, dn,
                                     preferred_element_type=jnp.float32,
                                     precision=lax.Precision.HIGHEST)
    cnt_ref[...] += lax.dot_general(onehot, jnp.ones((BLK, D), jnp.float32),
                                    dn, preferred_element_type=jnp.float32,
                                    precision=lax.Precision.HIGHEST)

    @pl.when(i == pl.num_programs(0) - 1)
    def _():
        hg = sums_ref[...] / jnp.maximum(cnt_ref[...], 1.0)
        o_ref[...] = jnp.dot(hg, cw_ref[...],
                             preferred_element_type=jnp.float32,
                             precision=lax.Precision.HIGHEST) + cb_ref[...]


def _mlp_pool(h, agg, batch2, W1, b1, W2, b2, cls_W, cls_b2):
    return pl.pallas_call(
        _mlp_pool_body,
        grid=(N // BLK,),
        in_specs=[
            pl.BlockSpec((BLK, D), lambda i: (i, 0)),
            pl.BlockSpec((NC, BLK, D), lambda i: (0, i, 0)),
            pl.BlockSpec((BLK, 1), lambda i: (i, 0)),
            pl.BlockSpec((D, H), lambda i: (0, 0)),
            pl.BlockSpec((1, H), lambda i: (0, 0)),
            pl.BlockSpec((H, D), lambda i: (0, 0)),
            pl.BlockSpec((1, D), lambda i: (0, 0)),
            pl.BlockSpec((D, C), lambda i: (0, 0)),
            pl.BlockSpec((1, C), lambda i: (0, 0)),
        ],
        out_specs=pl.BlockSpec((G, C), lambda i: (0, 0)),
        out_shape=jax.ShapeDtypeStruct((G, C), jnp.float32),
        scratch_shapes=[
            pltpu.VMEM((G, D), jnp.float32),
            pltpu.VMEM((G, D), jnp.float32),
        ],
    )(h, agg, batch2, W1, b1, W2, b2, cls_W, cls_b2)


def kernel(x, edge_index, batch,
           l0_W1, l0_b1, l0_bn_g, l0_bn_b, l0_W2, l0_b2, l0_obn_g, l0_obn_b,
           l1_W1, l1_b1, l1_bn_g, l1_bn_b, l1_W2, l1_b2, l1_obn_g, l1_obn_b,
           l2_W1, l2_b1, l2_bn_g, l2_bn_b, l2_W2, l2_b2, l2_obn_g, l2_obn_b,
           cls_W, cls_b):
    bscale = 1.0 / jnp.sqrt(jnp.float32(1.0 + 1e-5))
    layers = [
        (l0_W1, l0_b1, l0_bn_g, l0_bn_b, l0_W2, l0_b2, l0_obn_g, l0_obn_b),
        (l1_W1, l1_b1, l1_bn_g, l1_bn_b, l1_W2, l1_b2, l1_obn_g, l1_obn_b),
        (l2_W1, l2_b1, l2_bn_g, l2_bn_b, l2_W2, l2_b2, l2_obn_g, l2_obn_b),
    ]
    folded = []
    for (W1, b1, bg, bb, W2, b2, og, ob) in layers:
        s1 = bscale * bg
        s2 = bscale * og
        folded.append((W1 * s1[None, :], (b1 * s1 + bb)[None, :],
                       W2 * s2[None, :], (b2 * s2 + ob)[None, :]))

    pad = EPAD - E
    srcp = jnp.concatenate([edge_index[0],
                            jnp.zeros((pad,), jnp.int32)])
    dstp = jnp.concatenate([edge_index[1],
                            jnp.full((pad,), NPAD - 1, jnp.int32)])
    eidx = jnp.stack([srcp.reshape(NW, NCHUNK, CHUNK),
                      dstp.reshape(NW, NCHUNK, CHUNK)], axis=2)
    zeros = jnp.zeros((RPT, D), jnp.float32)
    batch2 = batch.reshape(N, 1)

    h = x
    for l in range(2):
        W1f, b1f, W2f, b2f = folded[l]
        agg = _segsum(h, eidx, zeros)
        h = _mlp(h, agg, W1f, b1f, W2f, b2f)
    W1f, b1f, W2f, b2f = folded[2]
    agg = _segsum(h, eidx, zeros)
    return _mlp_pool(h, agg, batch2, W1f, b1f, W2f, b2f,
                     cls_W, cls_b.reshape(1, C))


# speedup vs baseline: 1.0715x; 1.0715x over previous
import functools

import jax
import jax.numpy as jnp
from jax import lax
from jax.experimental import pallas as pl
from jax.experimental.pallas import tpu as pltpu, tpu_sc as plsc

N = 10000
E = 320000
D = 128
H = 256
G = 64
C = 10

NC = 2
NS = 16
NW = NC * NS
CHUNK = 80
NCHUNK = 128
EPW = NCHUNK * CHUNK
EPAD = NW * EPW
NBUF = 2
NGRP = NCHUNK // NBUF
NPAD = 10240
RPT = NPAD // NS


def _segsum_body(h_hbm, src_hbm, dst_hbm, zeros_hbm, out_hbm,
                 src_v, dst_v, rows_v, shared, *gsems):
    c = lax.axis_index("c")
    s = lax.axis_index("s")
    w = c * NS + s
    pltpu.sync_copy(zeros_hbm, shared.at[pl.ds(s * RPT, RPT)])
    pltpu.sync_copy(src_hbm.at[w], src_v)
    pltpu.sync_copy(dst_hbm.at[w], dst_v)
    for b in range(NBUF):
        pltpu.async_copy(h_hbm.at[src_v.at[pl.ds(b * CHUNK, CHUNK)]],
                         rows_v.at[b], gsems[b])
    plsc.subcore_barrier()

    def group(g, carry):
        for b in range(NBUF):
            j = g * NBUF + b
            pltpu.make_async_copy(
                h_hbm.at[src_v.at[pl.ds(j * CHUNK, CHUNK)]],
                rows_v.at[b], gsems[b]).wait()
            pltpu.sync_copy(rows_v.at[b], shared.at[dst_v.at[j]], add=True)

            @pl.when(j + NBUF < NCHUNK)
            def _():
                pltpu.async_copy(
                    h_hbm.at[src_v.at[pl.ds((j + NBUF) * CHUNK, CHUNK)]],
                    rows_v.at[b], gsems[b])
        return carry

    lax.fori_loop(0, NGRP, group, 0)
    plsc.subcore_barrier()
    pltpu.sync_copy(shared.at[pl.ds(s * RPT, RPT)],
                    out_hbm.at[c, pl.ds(s * RPT, RPT)])


def _segsum(h, src2, dst3, zeros):
    mesh = plsc.VectorSubcoreMesh(core_axis_name="c", subcore_axis_name="s")
    f = pl.kernel(
        _segsum_body,
        mesh=mesh,
        out_type=jax.ShapeDtypeStruct((NC, NPAD, D), jnp.float32),
        scratch_types=[
            pltpu.VMEM((EPW,), jnp.int32),
            pltpu.VMEM((NCHUNK, CHUNK), jnp.int32),
            pltpu.VMEM((NBUF, CHUNK, D), jnp.float32),
            pltpu.VMEM_SHARED((NPAD, D), jnp.float32),
        ] + [pltpu.SemaphoreType.DMA] * NBUF,
    )
    return f(h, src2, dst3, zeros)


BLK = 2000


def _mlp_compute(h_ref, a_ref, W1_ref, b1_ref, W2_ref, b2_ref, relu_out):
    z = h_ref[...] + a_ref[0] + a_ref[1]
    z = jnp.dot(z, W1_ref[...], preferred_element_type=jnp.float32,
                precision=lax.Precision.HIGHEST) + b1_ref[...]
    z = jnp.maximum(z, 0.0)
    z = jnp.dot(z, W2_ref[...], preferred_element_type=jnp.float32,
                precision=lax.Precision.HIGHEST) + b2_ref[...]
    if relu_out:
        z = jnp.maximum(z, 0.0)
    return z


def _mlp_body(h_ref, a_ref, W1_ref, b1_ref, W2_ref, b2_ref, o_ref):
    o_ref[...] = _mlp_compute(h_ref, a_ref, W1_ref, b1_ref, W2_ref, b2_ref,
                              relu_out=True)


def _mlp(h, agg, W1, b1, W2, b2):
    return pl.pallas_call(
        _mlp_body,
        grid=(N // BLK,),
        in_specs=[
            pl.BlockSpec((BLK, D), lambda i: (i, 0)),
            pl.BlockSpec((NC, BLK, D), lambda i: (0, i, 0)),
            pl.BlockSpec((D, H), lambda i: (0, 0)),
            pl.BlockSpec((1, H), lambda i: (0, 0)),
            pl.BlockSpec((H, D), lambda i: (0, 0)),
            pl.BlockSpec((1, D), lambda i: (0, 0)),
        ],
        out_specs=pl.BlockSpec((BLK, D), lambda i: (i, 0)),
        out_shape=jax.ShapeDtypeStruct((N, D), jnp.float32),
    )(h, agg, W1, b1, W2, b2)


def _mlp_pool_body(h_ref, a_ref, batch_ref, W1_ref, b1_ref, W2_ref, b2_ref,
                   cw_ref, cb_ref, o_ref, sums_ref, cnt_ref):
    i = pl.program_id(0)

    @pl.when(i == 0)
    def _():
        sums_ref[...] = jnp.zeros_like(sums_ref)
        cnt_ref[...] = jnp.zeros_like(cnt_ref)

    z = _mlp_compute(h_ref, a_ref, W1_ref, b1_ref, W2_ref, b2_ref,
                     relu_out=False)
    onehot = (batch_ref[...] ==
              lax.broadcasted_iota(jnp.int32, (BLK, G), 1)).astype(jnp.float32)
    dn = (((0,), (0,)), ((), ()))
    sums_ref[...] += lax.dot_general(onehot, z, dn,
                                     preferred_element_type=jnp.float32,
                                     precision=lax.Precision.HIGHEST)
    cnt_ref[...] += lax.dot_general(onehot, jnp.ones((BLK, D), jnp.float32),
                                    dn, preferred_element_type=jnp.float32,
                                    precision=lax.Precision.HIGHEST)

    @pl.when(i == pl.num_programs(0) - 1)
    def _():
        hg = sums_ref[...] / jnp.maximum(cnt_ref[...], 1.0)
        o_ref[...] = jnp.dot(hg, cw_ref[...],
                             preferred_element_type=jnp.float32,
                             precision=lax.Precision.HIGHEST) + cb_ref[...]


def _mlp_pool(h, agg, batch2, W1, b1, W2, b2, cls_W, cls_b2):
    return pl.pallas_call(
        _mlp_pool_body,
        grid=(N // BLK,),
        in_specs=[
            pl.BlockSpec((BLK, D), lambda i: (i, 0)),
            pl.BlockSpec((NC, BLK, D), lambda i: (0, i, 0)),
            pl.BlockSpec((BLK, 1), lambda i: (i, 0)),
            pl.BlockSpec((D, H), lambda i: (0, 0)),
            pl.BlockSpec((1, H), lambda i: (0, 0)),
            pl.BlockSpec((H, D), lambda i: (0, 0)),
            pl.BlockSpec((1, D), lambda i: (0, 0)),
            pl.BlockSpec((D, C), lambda i: (0, 0)),
            pl.BlockSpec((1, C), lambda i: (0, 0)),
        ],
        out_specs=pl.BlockSpec((G, C), lambda i: (0, 0)),
        out_shape=jax.ShapeDtypeStruct((G, C), jnp.float32),
        scratch_shapes=[
            pltpu.VMEM((G, D), jnp.float32),
            pltpu.VMEM((G, D), jnp.float32),
        ],
    )(h, agg, batch2, W1, b1, W2, b2, cls_W, cls_b2)


def kernel(x, edge_index, batch,
           l0_W1, l0_b1, l0_bn_g, l0_bn_b, l0_W2, l0_b2, l0_obn_g, l0_obn_b,
           l1_W1, l1_b1, l1_bn_g, l1_bn_b, l1_W2, l1_b2, l1_obn_g, l1_obn_b,
           l2_W1, l2_b1, l2_bn_g, l2_bn_b, l2_W2, l2_b2, l2_obn_g, l2_obn_b,
           cls_W, cls_b):
    bscale = 1.0 / jnp.sqrt(jnp.float32(1.0 + 1e-5))
    layers = [
        (l0_W1, l0_b1, l0_bn_g, l0_bn_b, l0_W2, l0_b2, l0_obn_g, l0_obn_b),
        (l1_W1, l1_b1, l1_bn_g, l1_bn_b, l1_W2, l1_b2, l1_obn_g, l1_obn_b),
        (l2_W1, l2_b1, l2_bn_g, l2_bn_b, l2_W2, l2_b2, l2_obn_g, l2_obn_b),
    ]
    folded = []
    for (W1, b1, bg, bb, W2, b2, og, ob) in layers:
        s1 = bscale * bg
        s2 = bscale * og
        folded.append((W1 * s1[None, :], (b1 * s1 + bb)[None, :],
                       W2 * s2[None, :], (b2 * s2 + ob)[None, :]))

    pad = EPAD - E
    srcp = jnp.concatenate([edge_index[0],
                            jnp.zeros((pad,), jnp.int32)])
    dstp = jnp.concatenate([edge_index[1],
                            jnp.full((pad,), NPAD - 1, jnp.int32)])
    src2 = srcp.reshape(NW, EPW)
    dst3 = dstp.reshape(NW, NCHUNK, CHUNK)
    zeros = jnp.zeros((RPT, D), jnp.float32)
    batch2 = batch.reshape(N, 1)

    h = x
    for l in range(2):
        W1f, b1f, W2f, b2f = folded[l]
        agg = _segsum(h, src2, dst3, zeros)
        h = _mlp(h, agg, W1f, b1f, W2f, b2f)
    W1f, b1f, W2f, b2f = folded[2]
    agg = _segsum(h, src2, dst3, zeros)
    return _mlp_pool(h, agg, batch2, W1f, b1f, W2f, b2f,
                     cls_W, cls_b.reshape(1, C))
